# Initial kernel scaffold; baseline (speedup 1.0000x reference)
import jax
import jax.numpy as jnp
from jax.experimental import pallas as pl


def kernel(tokens, word_weight):
    raise NotImplementedError("write your pallas kernel here")


# speedup vs baseline: 1.3071x; 1.3071x over previous
import functools

import jax
import jax.numpy as jnp
from jax import lax
from jax.experimental import pallas as pl
from jax.experimental.pallas import tpu as pltpu
from jax.experimental.pallas import tpu_sc as plsc

VOCAB = 1000000
DIM = 32
B = 4096
L = 200

NC = 2
NS = 16
NW = NC * NS
NTOK = B * L
PER_W = NTOK // NW
CHUNK = 128
NCHUNK = PER_W // CHUNK


def _body(tok_hbm, table_hbm, out_hbm, idx_v, buf, sem):
    wid = lax.axis_index("s") * NC + lax.axis_index("c")
    base = wid * PER_W
    pltpu.sync_copy(tok_hbm.at[wid], idx_v)

    @pl.loop(0, NCHUNK)
    def _chunk(j):
        pltpu.async_copy(table_hbm.at[idx_v.at[j]], buf, sem).wait()
        pltpu.sync_copy(buf, out_hbm.at[pl.ds(base + j * CHUNK, CHUNK)])


@functools.partial(jax.jit, static_argnames=())
def kernel(tokens, word_weight):
    tok = tokens.reshape(NW, NCHUNK, CHUNK).astype(jnp.int32)
    grid_kernel = pl.kernel(
        _body,
        out_type=jax.ShapeDtypeStruct((NTOK, DIM), jnp.float32),
        mesh=plsc.VectorSubcoreMesh(core_axis_name="c", subcore_axis_name="s"),
        scratch_types=[
            pltpu.VMEM((NCHUNK, CHUNK), jnp.int32),
            pltpu.VMEM((CHUNK, DIM), jnp.float32),
            pltpu.SemaphoreType.DMA,
        ],
        compiler_params=pltpu.CompilerParams(use_tc_tiling_on_sc=False),
    )
    out = grid_kernel(tok, word_weight)
    return out.reshape(B, L, DIM)


# speedup vs baseline: 1.4984x; 1.1464x over previous
import functools

import jax
import jax.numpy as jnp
from jax import lax
from jax.experimental import pallas as pl
from jax.experimental.pallas import tpu as pltpu
from jax.experimental.pallas import tpu_sc as plsc

VOCAB = 1000000
DIM = 32
B = 4096
L = 200

NC = 2
NS = 16
NW = NC * NS
NTOK = B * L
PER_W = NTOK // NW
CHUNK = 512
NCHUNK = PER_W // CHUNK


def _body(tok_hbm, table_hbm, out_hbm, idx_v, buf0, buf1, gsem0, gsem1,
          wsem0, wsem1):
    wid = lax.axis_index("s") * NC + lax.axis_index("c")
    bufs = (buf0, buf1)
    gsems = (gsem0, gsem1)
    wsems = (wsem0, wsem1)
    pltpu.sync_copy(tok_hbm.at[wid], idx_v)

    def gather(c, b):
        pltpu.async_copy(table_hbm.at[idx_v.at[c]], bufs[b], gsems[b])

    def write(c, b):
        base = wid * PER_W + c * CHUNK
        pltpu.async_copy(bufs[b], out_hbm.at[pl.ds(base, CHUNK)], wsems[b])

    gather(0, 0)
    gather(1, 1)

    @pl.loop(0, NCHUNK, step=2)
    def _pair(g):
        for b in range(2):
            c = g + b
            pltpu.make_async_copy(table_hbm.at[idx_v.at[c]], bufs[b],
                                  gsems[b]).wait()
            write(c, b)

            @pl.when(c + 2 < NCHUNK)
            def _():
                pltpu.make_async_copy(bufs[b], out_hbm.at[pl.ds(0, CHUNK)],
                                      wsems[b]).wait()
                gather(c + 2, b)

    for b in range(2):
        pltpu.make_async_copy(bufs[b], out_hbm.at[pl.ds(0, CHUNK)],
                              wsems[b]).wait()


@functools.partial(jax.jit, static_argnames=())
def kernel(tokens, word_weight):
    tok = tokens.reshape(NW, NCHUNK, CHUNK).astype(jnp.int32)
    grid_kernel = pl.kernel(
        _body,
        out_type=jax.ShapeDtypeStruct((NTOK, DIM), jnp.float32),
        mesh=plsc.VectorSubcoreMesh(core_axis_name="c", subcore_axis_name="s"),
        scratch_types=[
            pltpu.VMEM((NCHUNK, CHUNK), jnp.int32),
            pltpu.VMEM((CHUNK, DIM), jnp.float32),
            pltpu.VMEM((CHUNK, DIM), jnp.float32),
            pltpu.SemaphoreType.DMA,
            pltpu.SemaphoreType.DMA,
            pltpu.SemaphoreType.DMA,
            pltpu.SemaphoreType.DMA,
        ],
        compiler_params=pltpu.CompilerParams(use_tc_tiling_on_sc=False),
    )
    out = grid_kernel(tok, word_weight)
    return out.reshape(B, L, DIM)
